# baseline (speedup 1.0000x reference)
import functools

import jax
import jax.numpy as jnp
from jax import lax
from jax.experimental import pallas as pl
from jax.experimental.pallas import tpu as pltpu
from jax.experimental.pallas import tpu_sc as plsc

B = 16384
V = 49
VP = 64
DE = 3
C = 4
DO = 3
L = 16
NW = 32
BW = B // NW

_mesh = plsc.VectorSubcoreMesh(core_axis_name="c", subcore_axis_name="s")


@functools.partial(
    pl.kernel,
    out_type=(
        jax.ShapeDtypeStruct((B * DO,), jnp.float32),
        jax.ShapeDtypeStruct((B * DE,), jnp.float32),
    ),
    mesh=_mesh,
    compiler_params=pltpu.CompilerParams(
        needs_layout_passes=False, use_tc_tiling_on_sc=False
    ),
    scratch_types=[
        pltpu.VMEM((BW * C,), jnp.int32),
        pltpu.VMEM((BW,), jnp.int32),
        pltpu.VMEM((VP * DE,), jnp.float32),
        pltpu.VMEM((48,), jnp.float32),
        pltpu.VMEM((L,), jnp.float32),
        pltpu.VMEM((C * VP * DO,), jnp.float32),
        pltpu.VMEM((BW * DO,), jnp.float32),
        pltpu.VMEM((BW * DE,), jnp.float32),
    ],
)
def _cbow_sc(x_hbm, x1_hbm, emb_hbm, w_hbm, b_hbm, y_hbm, y1_hbm,
             x_v, x1_v, emb_v, w_v, b_v, t_v, y_v, y1_v):
    nc = _mesh.num_cores
    wid = lax.axis_index("s") * nc + lax.axis_index("c")

    pltpu.sync_copy(x_hbm.at[pl.ds(wid * (BW * C), BW * C)], x_v)
    pltpu.sync_copy(x1_hbm.at[pl.ds(wid * BW, BW)], x1_v)
    pltpu.sync_copy(emb_hbm, emb_v.at[pl.ds(0, V * DE)])
    pltpu.sync_copy(w_hbm, w_v)
    pltpu.sync_copy(b_hbm, b_v)

    iota = lax.iota(jnp.int32, L)

    w_vec = [w_v[pl.ds(k * L, L)] for k in range(3)]
    b_vec = b_v[...]

    def w_scalar(j, k):
        idx = 12 * j + k
        return w_vec[idx // L][idx % L]

    for vg in range(VP // L):
        vv = vg * L + iota
        vv3 = vv * DE
        m = vv < V
        e = [plsc.load_gather(emb_v, [vv3 + d], mask=m) for d in range(DE)]
        for c in range(C):
            for j in range(DO):
                acc = e[0] * w_scalar(j, 3 * c + 0)
                acc = acc + e[1] * w_scalar(j, 3 * c + 1)
                acc = acc + e[2] * w_scalar(j, 3 * c + 2)
                if c == 0:
                    acc = acc + b_vec[j]
                plsc.store_scatter(t_v, [(c * VP * DO) + vv3 + j], acc, mask=m)

    def group(g, carry):
        rows = g * L + iota
        rows4 = rows * C
        rows3 = rows * DO
        xc = [plsc.load_gather(x_v, [rows4 + c]) for c in range(C)]
        tidx = [xc[c] * DO + (c * VP * DO) for c in range(C)]
        for j in range(DO):
            acc = plsc.load_gather(t_v, [tidx[0] + j])
            for c in range(1, C):
                acc = acc + plsc.load_gather(t_v, [tidx[c] + j])
            plsc.store_scatter(y_v, [rows3 + j], acc)
        x1c3 = x1_v[pl.ds(g * L, L)] * DE
        for j in range(DE):
            plsc.store_scatter(
                y1_v, [rows3 + j], plsc.load_gather(emb_v, [x1c3 + j])
            )
        return carry

    lax.fori_loop(0, BW // L, group, 0)

    pltpu.sync_copy(y_v, y_hbm.at[pl.ds(wid * (BW * DO), BW * DO)])
    pltpu.sync_copy(y1_v, y1_hbm.at[pl.ds(wid * (BW * DE), BW * DE)])


def kernel(x, x1, emb, W, b):
    w_flat = jnp.pad(W.reshape(-1), (0, 48 - C * DO * DE))
    b_pad = jnp.pad(b, (0, L - DO))
    y_lin, y1_lin = _cbow_sc(
        x.astype(jnp.int32).reshape(-1),
        x1.astype(jnp.int32),
        emb.reshape(-1),
        w_flat,
        b_pad,
    )
    return (y_lin.reshape(B, DO), y1_lin.reshape(B, DE))


# speedup vs baseline: 1.2827x; 1.2827x over previous
import functools

import jax
import jax.numpy as jnp
from jax import lax
from jax.experimental import pallas as pl
from jax.experimental.pallas import tpu as pltpu
from jax.experimental.pallas import tpu_sc as plsc

B = 16384
V = 49
VP = 64
DE = 3
C = 4
DO = 3
L = 16
NW = 32
BW = B // NW
CH = 128

_mesh = plsc.VectorSubcoreMesh(core_axis_name="c", subcore_axis_name="s")


@functools.partial(
    pl.kernel,
    out_type=(
        jax.ShapeDtypeStruct((B, DO), jnp.float32),
        jax.ShapeDtypeStruct((B, DE), jnp.float32),
    ),
    mesh=_mesh,
    compiler_params=pltpu.CompilerParams(
        needs_layout_passes=False, use_tc_tiling_on_sc=True
    ),
    scratch_types=[
        pltpu.VMEM((CH, C), jnp.int32),
        pltpu.VMEM((BW,), jnp.int32),
        pltpu.VMEM((VP * DE,), jnp.float32),
        pltpu.VMEM((48,), jnp.float32),
        pltpu.VMEM((L,), jnp.float32),
        pltpu.VMEM((C * VP * DO,), jnp.float32),
        pltpu.VMEM((CH, DO), jnp.float32),
        pltpu.VMEM((CH, DE), jnp.float32),
    ],
)
def _cbow_sc_t(x_hbm, x1_hbm, emb_hbm, w_hbm, b_hbm, y_hbm, y1_hbm,
               x_v, x1_v, emb_v, w_v, b_v, t_v, y_v, y1_v):
    nc = _mesh.num_cores
    wid = lax.axis_index("s") * nc + lax.axis_index("c")
    base = wid * BW

    pltpu.sync_copy(x1_hbm.at[pl.ds(base, BW)], x1_v)
    pltpu.sync_copy(emb_hbm, emb_v.at[pl.ds(0, V * DE)])
    pltpu.sync_copy(w_hbm, w_v)
    pltpu.sync_copy(b_hbm, b_v)

    iota = lax.iota(jnp.int32, L)
    w_vec = [w_v[pl.ds(k * L, L)] for k in range(3)]
    b_vec = b_v[...]

    def w_scalar(j, k):
        idx = 12 * j + k
        return w_vec[idx // L][idx % L]

    for vg in range(VP // L):
        vv = vg * L + iota
        vv3 = vv * DE
        m = vv < V
        e = [plsc.load_gather(emb_v, [vv3 + d], mask=m) for d in range(DE)]
        for c in range(C):
            for j in range(DO):
                acc = e[0] * w_scalar(j, 3 * c + 0)
                acc = acc + e[1] * w_scalar(j, 3 * c + 1)
                acc = acc + e[2] * w_scalar(j, 3 * c + 2)
                if c == 0:
                    acc = acc + b_vec[j]
                plsc.store_scatter(t_v, [(c * VP * DO) + vv3 + j], acc, mask=m)

    cconst = [jnp.full((L,), c, jnp.int32) for c in range(C)]
    jconst = [jnp.full((L,), j, jnp.int32) for j in range(DO)]

    def chunk(ch, carry):
        pltpu.sync_copy(x_hbm.at[pl.ds(base + ch * CH, CH), :], x_v)

        def group(g, carry2):
            rows = g * L + iota
            rows3 = rows * DO
            xc = [plsc.load_gather(x_v, [rows, cconst[c]]) for c in range(C)]
            tidx = [xc[c] * DO + (c * VP * DO) for c in range(C)]
            for j in range(DO):
                acc = plsc.load_gather(t_v, [tidx[0] + j])
                for c in range(1, C):
                    acc = acc + plsc.load_gather(t_v, [tidx[c] + j])
                plsc.store_scatter(y_v, [rows, jconst[j]], acc)
            x1c3 = x1_v[pl.ds(ch * CH + g * L, L)] * DE
            for j in range(DE):
                plsc.store_scatter(
                    y1_v, [rows, jconst[j]], plsc.load_gather(emb_v, [x1c3 + j])
                )
            return carry2

        lax.fori_loop(0, CH // L, group, 0)
        pltpu.sync_copy(y_v, y_hbm.at[pl.ds(base + ch * CH, CH), :])
        pltpu.sync_copy(y1_v, y1_hbm.at[pl.ds(base + ch * CH, CH), :])
        return carry

    lax.fori_loop(0, BW // CH, chunk, 0)


def kernel(x, x1, emb, W, b):
    w_flat = jnp.pad(W.reshape(-1), (0, 48 - C * DO * DE))
    b_pad = jnp.pad(b, (0, L - DO))
    return _cbow_sc_t(
        x.astype(jnp.int32), x1.astype(jnp.int32), emb.reshape(-1), w_flat, b_pad
    )


# speedup vs baseline: 2.7133x; 2.1152x over previous
import functools

import jax
import jax.numpy as jnp
from jax import lax
from jax.experimental import pallas as pl
from jax.experimental.pallas import tpu as pltpu
from jax.experimental.pallas import tpu_sc as plsc

B = 16384
V = 49
VP = 64
DE = 3
C = 4
DO = 3
L = 16
NW = 32
BW = B // NW

_mesh = plsc.VectorSubcoreMesh(core_axis_name="c", subcore_axis_name="s")


@functools.partial(
    pl.kernel,
    out_type=(
        jax.ShapeDtypeStruct((DO * B,), jnp.float32),
        jax.ShapeDtypeStruct((DE * B,), jnp.float32),
    ),
    mesh=_mesh,
    compiler_params=pltpu.CompilerParams(
        needs_layout_passes=False, use_tc_tiling_on_sc=False
    ),
    scratch_types=[
        pltpu.VMEM((C * BW,), jnp.int32),
        pltpu.VMEM((BW,), jnp.int32),
        pltpu.VMEM((176,), jnp.float32),
        pltpu.VMEM((48,), jnp.float32),
        pltpu.VMEM((L,), jnp.float32),
        pltpu.VMEM((C * DO * VP,), jnp.float32),
        pltpu.VMEM((DO * BW,), jnp.float32),
        pltpu.VMEM((DE * BW,), jnp.float32),
        pltpu.SemaphoreType.DMA,
        pltpu.SemaphoreType.DMA,
    ],
)
def _cbow_sc(xt_hbm, x1_hbm, embt_hbm, w_hbm, b_hbm, yt_hbm, y1t_hbm,
             x_v, x1_v, embt_v, w_v, b_v, t_v, yt_v, y1t_v, sem_in, sem_out):
    nc = _mesh.num_cores
    wid = lax.axis_index("s") * nc + lax.axis_index("c")
    base = wid * BW

    copies = [
        pltpu.async_copy(xt_hbm.at[pl.ds(c * B + base, BW)],
                         x_v.at[pl.ds(c * BW, BW)], sem_in)
        for c in range(C)
    ]
    copies.append(pltpu.async_copy(x1_hbm.at[pl.ds(base, BW)], x1_v, sem_in))
    copies.append(pltpu.async_copy(embt_hbm, embt_v.at[pl.ds(0, DE * V)], sem_in))
    copies.append(pltpu.async_copy(w_hbm, w_v.at[pl.ds(0, 36)], sem_in))
    copies.append(pltpu.async_copy(b_hbm, b_v.at[pl.ds(0, DO)], sem_in))
    for cp in copies:
        cp.wait()

    iota = lax.iota(jnp.int32, L)
    w_vec = [w_v[pl.ds(k * L, L)] for k in range(3)]
    b_vec = b_v[...]

    def w_scalar(j, k):
        idx = 12 * j + k
        return w_vec[idx // L][idx % L]

    for vg in range(VP // L):
        vv = vg * L + iota
        e = [plsc.load_gather(embt_v, [vv + d * V]) for d in range(DE)]
        for c in range(C):
            for j in range(DO):
                acc = e[0] * w_scalar(j, 3 * c + 0)
                acc = acc + e[1] * w_scalar(j, 3 * c + 1)
                acc = acc + e[2] * w_scalar(j, 3 * c + 2)
                if c == 0:
                    acc = acc + b_vec[j]
                t_v[pl.ds(c * (DO * VP) + j * VP + vg * L, L)] = acc

    def group(g, carry):
        xoff = g * L
        xc = [x_v[pl.ds(c * BW + xoff, L)] for c in range(C)]
        for j in range(DO):
            acc = plsc.load_gather(t_v, [xc[0] + (j * VP)])
            for c in range(1, C):
                acc = acc + plsc.load_gather(t_v, [xc[c] + (c * (DO * VP) + j * VP)])
            yt_v[pl.ds(j * BW + xoff, L)] = acc
        x1c = x1_v[pl.ds(xoff, L)]
        for j in range(DE):
            y1t_v[pl.ds(j * BW + xoff, L)] = plsc.load_gather(
                embt_v, [x1c + j * V]
            )
        return carry

    lax.fori_loop(0, BW // L, group, 0)

    out_copies = [
        pltpu.async_copy(yt_v.at[pl.ds(j * BW, BW)],
                         yt_hbm.at[pl.ds(j * B + base, BW)], sem_out)
        for j in range(DO)
    ]
    out_copies += [
        pltpu.async_copy(y1t_v.at[pl.ds(j * BW, BW)],
                         y1t_hbm.at[pl.ds(j * B + base, BW)], sem_out)
        for j in range(DE)
    ]
    for cp in out_copies:
        cp.wait()


def kernel(x, x1, emb, W, b):
    xt_flat = x.astype(jnp.int32).T.reshape(-1)
    embt_flat = emb.T.reshape(-1)
    w_flat = W.reshape(-1)
    yt, y1t = _cbow_sc(xt_flat, x1.astype(jnp.int32), embt_flat, w_flat, b)
    return (yt.reshape(DO, B).T, y1t.reshape(DE, B).T)
